# baseline (speedup 1.0000x reference)
import jax
import jax.numpy as jnp
from jax import lax
from jax.experimental import pallas as pl
from jax.experimental.pallas import tpu as pltpu

_DELTA_VAR = 0.5
_DELTA_DIST = 1.5
_ALPHA = 1.0
_BETA = 1.0
_GAMMA = 0.1
_MAX_ID = 8


def _body(emb_ref, mask_ref, maskf_ref, out_ref, acc_ref, v_ref, s_ref, t_ref):
    b = pl.program_id(0)
    nb = pl.num_programs(0)
    P = v_ref.shape[1]

    e0 = emb_ref[0, 0]
    e1 = emb_ref[0, 1]
    e2 = emb_ref[0, 2]
    e3 = emb_ref[0, 3]
    m = mask_ref[0]
    mf = maskf_ref[0]

    normsq = e0 * e0 + e1 * e1 + e2 * e2 + e3 * e3
    norm = jnp.sqrt(normsq)

    validf = (m != 0).astype(jnp.float32)
    n_valid = jnp.sum(validf)
    reg_sum = jnp.sum(norm * validf)

    v_ref[0:4, :] = emb_ref[0].reshape(4, P)

    @pl.when(b == 0)
    def _ones():
        v_ref[4:5, :] = jnp.ones((1, P), jnp.float32)

    iot = lax.broadcasted_iota(jnp.int32, (_MAX_ID, 1), 0)
    onehot = (mf == iot).astype(jnp.float32)

    s_ref[:, :] = lax.dot_general(
        onehot, v_ref[:, :], (((1,), (1,)), ((), ())),
        preferred_element_type=jnp.float32,
    )

    cnts = []
    means = []
    for u in range(1, _MAX_ID):
        cnt = s_ref[u, 4]
        safe = jnp.maximum(cnt, 1.0)
        cnts.append(cnt)
        means.append(
            (s_ref[u, 0] / safe, s_ref[u, 1] / safe,
             s_ref[u, 2] / safe, s_ref[u, 3] / safe)
        )

    mc0 = jnp.zeros_like(e0)
    mc1 = jnp.zeros_like(e0)
    mc2 = jnp.zeros_like(e0)
    mc3 = jnp.zeros_like(e0)
    for u in range(1, _MAX_ID):
        sel = m == u
        mu = means[u - 1]
        mc0 = jnp.where(sel, mu[0], mc0)
        mc1 = jnp.where(sel, mu[1], mc1)
        mc2 = jnp.where(sel, mu[2], mc2)
        mc3 = jnp.where(sel, mu[3], mc3)
    d0 = e0 - mc0
    d1 = e1 - mc1
    d2 = e2 - mc2
    d3 = e3 - mc3
    dist = jnp.sqrt(d0 * d0 + d1 * d1 + d2 * d2 + d3 * d3)

    t_ref[:, :] = lax.dot_general(
        onehot, dist.reshape(1, P), (((1,), (1,)), ((), ())),
        preferred_element_type=jnp.float32,
    )

    num_instances = jnp.float32(0.0)
    var_sum = jnp.float32(0.0)
    for u in range(1, _MAX_ID):
        cnt = cnts[u - 1]
        present = cnt > 0.0
        mean_norm = t_ref[u, 0] / jnp.maximum(cnt, 1.0)
        term = jnp.maximum(mean_norm - _DELTA_VAR, 0.0) ** 2
        var_sum = var_sum + jnp.where(present, term, 0.0)
        num_instances = num_instances + present.astype(jnp.float32)
    var_loss = var_sum / jnp.maximum(num_instances, 1.0)

    diag_term = (jnp.float32(_DELTA_DIST) + jnp.float32(1e6)) ** 2
    dist_sum = jnp.float32(0.0)
    for u in range(_MAX_ID - 1):
        dist_sum = dist_sum + jnp.where(cnts[u] > 0.0, diag_term, 0.0)
    for u in range(_MAX_ID - 1):
        for v in range(u + 1, _MAX_ID - 1):
            mu = means[u]
            mv = means[v]
            pairsq = (
                (mu[0] - mv[0]) ** 2
                + (mu[1] - mv[1]) ** 2
                + (mu[2] - mv[2]) ** 2
                + (mu[3] - mv[3]) ** 2
            )
            pd = jnp.sqrt(pairsq)
            hinge = jnp.maximum(_DELTA_DIST - pd, 0.0) ** 2
            both = jnp.logical_and(cnts[u] > 0.0, cnts[v] > 0.0)
            dist_sum = dist_sum + 2.0 * jnp.where(both, hinge, 0.0)
    denom = num_instances * (num_instances - 1.0)
    dist_loss = jnp.where(
        num_instances > 1.0, dist_sum / jnp.maximum(denom, 1.0), 0.0
    )

    reg_loss = reg_sum / jnp.maximum(n_valid, 1.0)
    loss_b = _ALPHA * var_loss + _BETA * dist_loss + _GAMMA * reg_loss
    inc = (n_valid > 0.0).astype(jnp.float32)

    @pl.when(b == 0)
    def _init():
        acc_ref[0] = 0.0
        acc_ref[1] = 0.0

    acc_ref[0] += loss_b * inc
    acc_ref[1] += inc

    @pl.when(b == nb - 1)
    def _fin():
        s = acc_ref[0]
        n = acc_ref[1]
        total = jnp.where(n > 0.0, s / jnp.maximum(n, 1.0), 0.0)
        out_ref[:, :] = jnp.broadcast_to(total, (1, 1))


def kernel(embeddings, instance_mask):
    B, C, H, W = embeddings.shape
    P = H * W
    maskf = instance_mask.reshape(B, 1, P)
    out = pl.pallas_call(
        _body,
        grid=(B,),
        in_specs=[
            pl.BlockSpec((1, C, H, W), lambda b: (b, 0, 0, 0)),
            pl.BlockSpec((1, H, W), lambda b: (b, 0, 0)),
            pl.BlockSpec((1, 1, P), lambda b: (b, 0, 0)),
        ],
        out_specs=pl.BlockSpec((1, 1), lambda b: (0, 0)),
        out_shape=jax.ShapeDtypeStruct((1, 1), jnp.float32),
        scratch_shapes=[
            pltpu.SMEM((2,), jnp.float32),
            pltpu.VMEM((5, P), jnp.float32),
            pltpu.VMEM((8, 5), jnp.float32),
            pltpu.VMEM((8, 1), jnp.float32),
        ],
    )(embeddings, instance_mask, maskf)
    return out[0, 0]
